# baseline (speedup 1.0000x reference)
import jax
import jax.numpy as jnp
from jax import lax
from jax.experimental import pallas as pl
from jax.experimental.pallas import tpu as pltpu
from jax.experimental.pallas import tpu_sc as plsc

D = 256
NX = NY = NZ = 32
B = 4
SC_B0 = 2
NC = 2
NS = 16
YC = 4
NCHUNK = NY // YC
NBUF = 2
NL = 16

XBLK = 8
TC_NSTEP = NX // XBLK
TC_NBUF = 2


def _sc_body(xe_hbm, ye_hbm, ze_hbm, out_hbm, xev, yev, zev, xey, buf, sems):
    wid = lax.axis_index("s") * NC + lax.axis_index("c")
    x = wid
    nb = B - SC_B0

    pltpu.sync_copy(xe_hbm.at[x], xev)
    pltpu.sync_copy(ye_hbm.at[pl.ds(0, NY)], yev)
    pltpu.sync_copy(ze_hbm.at[pl.ds(0, NZ)], zev)

    for j in range(NCHUNK):
        slot = j % NBUF
        if j >= NBUF:
            for bb in range(nb):
                pltpu.make_async_copy(
                    buf.at[slot],
                    out_hbm.at[SC_B0 + bb, x, pl.ds((j - NBUF) * YC, YC)],
                    sems.at[slot, bb]).wait()

        for yy in range(YC):
            for c in range(D // NL):
                sl = pl.ds(c * NL, NL)
                xey[yy, sl] = xev[sl] + yev[j * YC + yy, sl]

        for c in range(D // NL):
            sl = pl.ds(c * NL, NL)
            xv = [xey[yy, sl] for yy in range(YC)]

            def z_step(z, _, sl=sl, xv=xv):
                zv = zev[z, sl]
                for yy in range(YC):
                    buf[slot, yy, z, sl] = xv[yy] + zv
                return 0

            lax.fori_loop(0, NZ, z_step, 0)

        for bb in range(nb):
            pltpu.make_async_copy(
                buf.at[slot],
                out_hbm.at[SC_B0 + bb, x, pl.ds(j * YC, YC)],
                sems.at[slot, bb]).start()

    for j in range(NCHUNK - NBUF, NCHUNK):
        slot = j % NBUF
        for bb in range(B - SC_B0):
            pltpu.make_async_copy(
                buf.at[slot],
                out_hbm.at[SC_B0 + bb, x, pl.ds(j * YC, YC)],
                sems.at[slot, bb]).wait()


@jax.jit
def _sc_call(xe, ye, ze):
    mesh = plsc.VectorSubcoreMesh(core_axis_name="c", subcore_axis_name="s")
    return pl.kernel(
        _sc_body,
        out_type=jax.ShapeDtypeStruct((B, NX, NY, NZ, D), jnp.float32),
        mesh=mesh,
        scratch_types=[
            pltpu.VMEM((D,), jnp.float32),
            pltpu.VMEM((NY, D), jnp.float32),
            pltpu.VMEM((NZ, D), jnp.float32),
            pltpu.VMEM((YC, D), jnp.float32),
            pltpu.VMEM((NBUF, YC, NZ, D), jnp.float32),
            pltpu.SemaphoreType.DMA((NBUF, B - SC_B0)),
        ],
    )(xe, ye, ze)


def _tc_body(xe_ref, ye_ref, ze_ref, big_ref, out_ref, scratch, sems):
    i = pl.program_id(0)
    slot = jax.lax.rem(i, TC_NBUF)

    xe = xe_ref[...]
    ye = ye_ref[...]
    ze = ze_ref[...]
    yz = ye[:, None, :] + ze[None, :, :]
    pos = xe[:, None, None, :] + yz[None]

    for k in range(TC_NBUF):
        @pl.when(slot == k)
        def _():
            @pl.when(i >= TC_NBUF)
            def _():
                for bb in range(SC_B0):
                    pltpu.make_async_copy(
                        scratch.at[k], out_ref.at[bb, pl.ds((i - TC_NBUF) * XBLK, XBLK)],
                        sems.at[k, bb]).wait()

            scratch[k] = pos

            for bb in range(SC_B0):
                pltpu.make_async_copy(
                    scratch.at[k], out_ref.at[bb, pl.ds(i * XBLK, XBLK)],
                    sems.at[k, bb]).start()

    @pl.when(i == TC_NSTEP - 1)
    def _():
        for k in range(TC_NBUF):
            step = i - ((i - k) % TC_NBUF)
            for bb in range(SC_B0):
                pltpu.make_async_copy(
                    scratch.at[k], out_ref.at[bb, pl.ds(step * XBLK, XBLK)],
                    sems.at[k, bb]).wait()


def _tc_call(xe, ye, ze, big):
    return pl.pallas_call(
        _tc_body,
        grid=(TC_NSTEP,),
        in_specs=[
            pl.BlockSpec((XBLK, D), lambda i: (i, 0)),
            pl.BlockSpec((NY, D), lambda i: (0, 0)),
            pl.BlockSpec((NZ, D), lambda i: (0, 0)),
            pl.BlockSpec(memory_space=pl.ANY),
        ],
        out_specs=pl.BlockSpec(memory_space=pl.ANY),
        out_shape=jax.ShapeDtypeStruct((B, NX, NY, NZ, D), jnp.float32),
        scratch_shapes=[
            pltpu.VMEM((TC_NBUF, XBLK, NY, NZ, D), jnp.float32),
            pltpu.SemaphoreType.DMA((TC_NBUF, SC_B0)),
        ],
        input_output_aliases={3: 0},
    )(xe, ye, ze, big)


def kernel(features, x_embed, y_embed, z_embed):
    xe = x_embed[:NX]
    ye = y_embed[:NY]
    ze = z_embed[:NZ]
    big = _sc_call(x_embed, y_embed, z_embed)
    out = _tc_call(xe, ye, ze, big)
    return jnp.transpose(out, (0, 4, 1, 2, 3))


# speedup vs baseline: 1.0335x; 1.0335x over previous
import functools
import jax
import jax.numpy as jnp
from jax import lax
from jax.experimental import pallas as pl
from jax.experimental.pallas import tpu as pltpu
from jax.experimental.pallas import tpu_sc as plsc

D = 256
NX = NY = NZ = 32
B = 4
NC = 2
NS = 16
YC = 4
NCHUNK = NY // YC
NBUF = 3
NL = 16


def _body(xe_hbm, ye_hbm, ze_hbm, out_hbm, xev, yev, zev, xey, buf, sems):
    wid = lax.axis_index("s") * NC + lax.axis_index("c")
    x = wid

    pltpu.sync_copy(xe_hbm.at[x], xev)
    pltpu.sync_copy(ye_hbm.at[pl.ds(0, NY)], yev)
    pltpu.sync_copy(ze_hbm.at[pl.ds(0, NZ)], zev)

    for j in range(NCHUNK):
        slot = j % NBUF
        if j >= NBUF:
            for bb in range(B):
                pltpu.make_async_copy(
                    buf.at[slot],
                    out_hbm.at[bb, x, pl.ds((j - NBUF) * YC, YC)],
                    sems.at[slot, bb]).wait()

        for yy in range(YC):
            for c in range(D // NL):
                sl = pl.ds(c * NL, NL)
                xey[yy, sl] = xev[sl] + yev[j * YC + yy, sl]

        for c in range(D // NL):
            sl = pl.ds(c * NL, NL)
            xv = [xey[yy, sl] for yy in range(YC)]

            @plsc.parallel_loop(0, NZ, unroll=2)
            def z_step(z, sl=sl, xv=xv):
                zv = zev[z, sl]
                for yy in range(YC):
                    buf[slot, yy, z, sl] = xv[yy] + zv

        for bb in range(B):
            pltpu.make_async_copy(
                buf.at[slot],
                out_hbm.at[bb, x, pl.ds(j * YC, YC)],
                sems.at[slot, bb]).start()

    for j in range(NCHUNK - NBUF, NCHUNK):
        slot = j % NBUF
        for bb in range(B):
            pltpu.make_async_copy(
                buf.at[slot],
                out_hbm.at[bb, x, pl.ds(j * YC, YC)],
                sems.at[slot, bb]).wait()


@functools.partial(jax.jit, static_argnames=())
def _sc_call(xe, ye, ze):
    mesh = plsc.VectorSubcoreMesh(core_axis_name="c", subcore_axis_name="s")
    return pl.kernel(
        _body,
        out_type=jax.ShapeDtypeStruct((B, NX, NY, NZ, D), jnp.float32),
        mesh=mesh,
        scratch_types=[
            pltpu.VMEM((D,), jnp.float32),
            pltpu.VMEM((NY, D), jnp.float32),
            pltpu.VMEM((NZ, D), jnp.float32),
            pltpu.VMEM((YC, D), jnp.float32),
            pltpu.VMEM((NBUF, YC, NZ, D), jnp.float32),
            pltpu.SemaphoreType.DMA((NBUF, B)),
        ],
    )(xe, ye, ze)


def kernel(features, x_embed, y_embed, z_embed):
    out = _sc_call(x_embed, y_embed, z_embed)
    return jnp.transpose(out, (0, 4, 1, 2, 3))


# speedup vs baseline: 1.0356x; 1.0020x over previous
import functools
import jax
import jax.numpy as jnp
from jax import lax
from jax.experimental import pallas as pl
from jax.experimental.pallas import tpu as pltpu
from jax.experimental.pallas import tpu_sc as plsc

D = 256
NX = NY = NZ = 32
B = 4
NC = 2
NS = 16
YC = 4
NCHUNK = NY // YC
NBUF = 2
NL = 16


def _body(xe_hbm, ye_hbm, ze_hbm, out_hbm, xev, yev, zev, xey, buf, sems):
    wid = lax.axis_index("s") * NC + lax.axis_index("c")
    x = wid

    pltpu.sync_copy(xe_hbm.at[x], xev)
    pltpu.sync_copy(ye_hbm.at[pl.ds(0, NY)], yev)
    pltpu.sync_copy(ze_hbm.at[pl.ds(0, NZ)], zev)

    for j in range(NCHUNK):
        slot = j % NBUF
        if j >= NBUF:
            for bb in range(B):
                pltpu.make_async_copy(
                    buf.at[slot],
                    out_hbm.at[bb, x, pl.ds((j - NBUF) * YC, YC)],
                    sems.at[slot, bb]).wait()

        for yy in range(YC):
            for c in range(D // NL):
                sl = pl.ds(c * NL, NL)
                xey[yy, sl] = xev[sl] + yev[j * YC + yy, sl]

        for c in range(D // NL):
            sl = pl.ds(c * NL, NL)
            xv = [xey[yy, sl] for yy in range(YC)]

            @plsc.parallel_loop(0, NZ, unroll=2)
            def z_step(z, sl=sl, xv=xv):
                zv = zev[z, sl]
                for yy in range(YC):
                    buf[slot, yy, z, sl] = xv[yy] + zv

        for bb in range(B):
            pltpu.make_async_copy(
                buf.at[slot],
                out_hbm.at[bb, x, pl.ds(j * YC, YC)],
                sems.at[slot, bb]).start()

    for j in range(NCHUNK - NBUF, NCHUNK):
        slot = j % NBUF
        for bb in range(B):
            pltpu.make_async_copy(
                buf.at[slot],
                out_hbm.at[bb, x, pl.ds(j * YC, YC)],
                sems.at[slot, bb]).wait()


@functools.partial(jax.jit, static_argnames=())
def _sc_call(xe, ye, ze):
    mesh = plsc.VectorSubcoreMesh(core_axis_name="c", subcore_axis_name="s")
    return pl.kernel(
        _body,
        out_type=jax.ShapeDtypeStruct((B, NX, NY, NZ, D), jnp.float32),
        mesh=mesh,
        scratch_types=[
            pltpu.VMEM((D,), jnp.float32),
            pltpu.VMEM((NY, D), jnp.float32),
            pltpu.VMEM((NZ, D), jnp.float32),
            pltpu.VMEM((YC, D), jnp.float32),
            pltpu.VMEM((NBUF, YC, NZ, D), jnp.float32),
            pltpu.SemaphoreType.DMA((NBUF, B)),
        ],
    )(xe, ye, ze)


def kernel(features, x_embed, y_embed, z_embed):
    out = _sc_call(x_embed, y_embed, z_embed)
    return jnp.transpose(out, (0, 4, 1, 2, 3))


# speedup vs baseline: 1.0579x; 1.0215x over previous
import functools
import jax
import jax.numpy as jnp
from jax import lax
from jax.experimental import pallas as pl
from jax.experimental.pallas import tpu as pltpu
from jax.experimental.pallas import tpu_sc as plsc

D = 256
NX = NY = NZ = 32
B = 4
NC = 2
NS = 16
YC = 4
NCHUNK = NY // YC
NBUF = 2
NL = 16


def _body(xe_hbm, ye_hbm, ze_hbm, out_hbm, xev, yev, zev, xey, buf, sems):
    wid = lax.axis_index("s") * NC + lax.axis_index("c")
    x = wid

    pltpu.sync_copy(xe_hbm.at[x], xev)
    pltpu.sync_copy(ye_hbm.at[pl.ds(0, NY)], yev)
    pltpu.sync_copy(ze_hbm.at[pl.ds(0, NZ)], zev)

    for j in range(NCHUNK):
        slot = j % NBUF
        if j >= NBUF:
            for bb in range(B):
                pltpu.make_async_copy(
                    buf.at[slot],
                    out_hbm.at[bb, x, pl.ds((j - NBUF) * YC, YC)],
                    sems.at[slot, bb]).wait()

        for yy in range(YC):
            for c in range(D // NL):
                sl = pl.ds(c * NL, NL)
                xey[yy, sl] = xev[sl] + yev[j * YC + yy, sl]

        for c in range(D // NL):
            sl = pl.ds(c * NL, NL)
            xv = [xey[yy, sl] for yy in range(YC)]

            def z_step(z, _, sl=sl, xv=xv):
                zv = zev[z, sl]
                for yy in range(YC):
                    buf[slot, yy, z, sl] = xv[yy] + zv
                return 0

            lax.fori_loop(0, NZ, z_step, 0)

        for bb in range(B):
            pltpu.make_async_copy(
                buf.at[slot],
                out_hbm.at[bb, x, pl.ds(j * YC, YC)],
                sems.at[slot, bb]).start()

    for j in range(NCHUNK - NBUF, NCHUNK):
        slot = j % NBUF
        for bb in range(B):
            pltpu.make_async_copy(
                buf.at[slot],
                out_hbm.at[bb, x, pl.ds(j * YC, YC)],
                sems.at[slot, bb]).wait()


@functools.partial(jax.jit, static_argnames=())
def _sc_call(xe, ye, ze):
    mesh = plsc.VectorSubcoreMesh(core_axis_name="c", subcore_axis_name="s")
    return pl.kernel(
        _body,
        out_type=jax.ShapeDtypeStruct((B, NX, NY, NZ, D), jnp.float32),
        mesh=mesh,
        scratch_types=[
            pltpu.VMEM((D,), jnp.float32),
            pltpu.VMEM((NY, D), jnp.float32),
            pltpu.VMEM((NZ, D), jnp.float32),
            pltpu.VMEM((YC, D), jnp.float32),
            pltpu.VMEM((NBUF, YC, NZ, D), jnp.float32),
            pltpu.SemaphoreType.DMA((NBUF, B)),
        ],
    )(xe, ye, ze)


def kernel(features, x_embed, y_embed, z_embed):
    out = _sc_call(x_embed, y_embed, z_embed)
    return jnp.transpose(out, (0, 4, 1, 2, 3))
